# baseline (speedup 1.0000x reference)
import jax
import jax.numpy as jnp
from jax import lax
from jax.experimental import pallas as pl
from jax.experimental.pallas import tpu as pltpu
from jax.experimental.pallas import tpu_sc as plsc

D_MODEL = 64
SCALE = 8.0
NC, NS, LANES = 2, 16, 16
NW = NC * NS
CHUNK = 128
NBUF = 4
KGRP = D_MODEL // LANES


def _emb_body(x_hbm, table_hbm, out_hbm, idx_v, pidx, ibuf, sel, *sems):
    gsems = sems[:NBUF]
    ssems = sems[NBUF:]
    rpw = x_hbm.shape[0] // NW
    sid = lax.axis_index("s")
    wid = sid * NC + lax.axis_index("c")
    row0 = wid * rpw

    pltpu.sync_copy(x_hbm.at[pl.ds(row0, rpw)], idx_v)

    def fire_gather(cj, b):
        for g in range(CHUNK // LANES):
            iv = idx_v[cj, pl.ds(g * LANES, LANES)]
            pidx[b, pl.ds(g * LANES, LANES)] = iv >> 1
        pltpu.async_copy(table_hbm.at[pidx.at[b]], ibuf.at[b], gsems[b])

    def native_base(u):
        return 2048 * (u // 32) + 8 * (u % 32)

    for b in range(NBUF):
        fire_gather(b, b)

    @pl.loop(0, rpw, step=NBUF)
    def _(j):
        for b in range(NBUF):
            cj = j + b
            pltpu.make_async_copy(
                table_hbm.at[pidx.at[b]], ibuf.at[b], gsems[b]
            ).wait()

            @pl.when(cj >= NBUF)
            def _():
                pbase = native_base(row0 + cj - NBUF)
                for f in range(D_MODEL):
                    off = 256 * (f // 8) + (f % 8)
                    pltpu.make_async_copy(
                        sel.at[b, :, :, f],
                        out_hbm.at[pl.ds(pbase + off, 1)],
                        ssems[b],
                    ).wait()

            @pl.loop(0, CHUNK, step=LANES)
            def _(r):
                par_v = (idx_v[cj, pl.ds(r, LANES)] & 1) * D_MODEL
                for rr in range(LANES):
                    row = r + rr
                    half = par_v[rr]
                    for k in range(KGRP):
                        sel[b, 0, row, pl.ds(k * LANES, LANES)] = (
                            ibuf[b, row, pl.ds(half + k * LANES, LANES)] * SCALE
                        )

            base = native_base(row0 + cj)
            for f in range(D_MODEL):
                off = 256 * (f // 8) + (f % 8)
                pltpu.async_copy(
                    sel.at[b, :, :, f],
                    out_hbm.at[pl.ds(base + off, 1)],
                    ssems[b],
                )

            @pl.when(cj + NBUF < rpw)
            def _():
                fire_gather(cj + NBUF, b)

    for b in range(NBUF):
        pbase = native_base(row0 + rpw - NBUF + b)
        for f in range(D_MODEL):
            off = 256 * (f // 8) + (f % 8)
            pltpu.make_async_copy(
                sel.at[b, :, :, f],
                out_hbm.at[pl.ds(pbase + off, 1)],
                ssems[b],
            ).wait()


def kernel(x, table):
    b0, b1 = x.shape
    total = b0 * b1
    ntile = b0 // CHUNK
    xt = x.T.reshape(b1 * ntile, CHUNK)
    t2 = table.reshape(table.shape[0] // 2, 2 * D_MODEL)
    run = pl.kernel(
        _emb_body,
        out_type=jax.ShapeDtypeStruct((total // 2, 2 * D_MODEL), jnp.float32),
        mesh=plsc.VectorSubcoreMesh(core_axis_name="c", subcore_axis_name="s"),
        scratch_types=[
            pltpu.VMEM((total // CHUNK // NW, CHUNK), jnp.int32),
            pltpu.VMEM((NBUF, CHUNK), jnp.int32),
            pltpu.VMEM((NBUF, CHUNK, 2 * D_MODEL), jnp.float32),
            pltpu.VMEM((NBUF, 1, CHUNK, D_MODEL), jnp.float32),
        ]
        + [pltpu.SemaphoreType.DMA] * (2 * NBUF),
        compiler_params=pltpu.CompilerParams(use_tc_tiling_on_sc=False),
    )
    out2 = run(xt, t2)
    out = out2.reshape(b1, 8, ntile, 8, CHUNK).transpose(2, 4, 0, 1, 3)
    return out.reshape(b0, b1, D_MODEL)


# speedup vs baseline: 12.9277x; 12.9277x over previous
import jax
import jax.numpy as jnp
from jax import lax
from jax.experimental import pallas as pl
from jax.experimental.pallas import tpu as pltpu
from jax.experimental.pallas import tpu_sc as plsc

D_MODEL = 64
SCALE = 8.0
NC, NS, LANES = 2, 16, 16
NW = NC * NS
CHUNK = 128
NBUF = 4
KGRP = D_MODEL // LANES


def _emb_body(x_hbm, table_hbm, out_hbm, idx_v, pidx, ibuf, sel, *sems):
    gsems = sems[:NBUF]
    ssems = sems[NBUF:]
    rpw = x_hbm.shape[0] // NW
    sid = lax.axis_index("s")
    wid = sid * NC + lax.axis_index("c")
    row0 = wid * rpw

    pltpu.sync_copy(x_hbm.at[pl.ds(row0, rpw)], idx_v)

    def fire_gather(cj, b):
        for g in range(CHUNK // LANES):
            iv = idx_v[cj, pl.ds(g * LANES, LANES)]
            pidx[b, pl.ds(g * LANES, LANES)] = iv >> 1
        pltpu.async_copy(table_hbm.at[pidx.at[b]], ibuf.at[b], gsems[b])

    def native_base(u):
        return 2048 * (u // 32) + 8 * (u % 32)

    for b in range(NBUF):
        fire_gather(b, b)

    @pl.loop(0, rpw, step=NBUF)
    def _(j):
        for b in range(NBUF):
            cj = j + b
            pltpu.make_async_copy(
                table_hbm.at[pidx.at[b]], ibuf.at[b], gsems[b]
            ).wait()

            @pl.when(cj >= NBUF)
            def _():
                up = row0 + cj - NBUF
                pltpu.make_async_copy(
                    sel.at[b], out_hbm.at[pl.ds(up * 64, 64)], ssems[b]
                ).wait()

            @pl.loop(0, CHUNK, step=LANES)
            def _(r):
                par_v = (idx_v[cj, pl.ds(r, LANES)] & 1) * D_MODEL
                for rr in range(LANES):
                    row = r + rr
                    half = par_v[rr]
                    for k in range(KGRP):
                        dst = pl.ds((row % 2) * D_MODEL + k * LANES, LANES)
                        sel[b, row // 2, dst] = (
                            ibuf[b, row, pl.ds(half + k * LANES, LANES)] * SCALE
                        )

            u = row0 + cj
            pltpu.async_copy(
                sel.at[b], out_hbm.at[pl.ds(u * 64, 64)], ssems[b]
            )

            @pl.when(cj + NBUF < rpw)
            def _():
                fire_gather(cj + NBUF, b)

    for b in range(NBUF):
        up = row0 + rpw - NBUF + b
        pltpu.make_async_copy(
            sel.at[b], out_hbm.at[pl.ds(up * 64, 64)], ssems[b]
        ).wait()


def kernel(x, table):
    b0, b1 = x.shape
    total = b0 * b1
    ntile = b0 // CHUNK
    xt = x.T.reshape(b1 * ntile, CHUNK)
    t2 = table.reshape(table.shape[0] // 2, 2 * D_MODEL)
    run = pl.kernel(
        _emb_body,
        out_type=jax.ShapeDtypeStruct((total // 2, 2 * D_MODEL), jnp.float32),
        mesh=plsc.VectorSubcoreMesh(core_axis_name="c", subcore_axis_name="s"),
        scratch_types=[
            pltpu.VMEM((total // CHUNK // NW, CHUNK), jnp.int32),
            pltpu.VMEM((NBUF, CHUNK), jnp.int32),
            pltpu.VMEM((NBUF, CHUNK, 2 * D_MODEL), jnp.float32),
            pltpu.VMEM((NBUF, CHUNK // 2, 2 * D_MODEL), jnp.float32),
        ]
        + [pltpu.SemaphoreType.DMA] * (2 * NBUF),
        compiler_params=pltpu.CompilerParams(use_tc_tiling_on_sc=False),
    )
    out2 = run(xt, t2)

    out5 = pl.pallas_call(
        _unit_transpose_body,
        grid=(b1, ntile),
        in_specs=[
            pl.BlockSpec((1, D_MODEL, CHUNK), lambda j, t: (j * 32 + t, 0, 0))
        ],
        out_specs=pl.BlockSpec(
            (1, 8, 1, 8, CHUNK), lambda j, t: (j, 0, t, 0, 0)
        ),
        out_shape=jax.ShapeDtypeStruct((b1, 8, ntile, 8, CHUNK), jnp.float32),
    )(out2.reshape(b1 * ntile, D_MODEL, CHUNK))
    out = out5.transpose(2, 4, 0, 1, 3)
    return out.reshape(b0, b1, D_MODEL)


def _unit_transpose_body(in_ref, o_ref):
    data = in_ref[0]
    d3 = data.reshape(D_MODEL, 2, D_MODEL)
    t = d3.transpose(2, 0, 1)
    o_ref[0, :, 0, :, :] = t.reshape(8, 8, CHUNK)


# speedup vs baseline: 59.4084x; 4.5954x over previous
import jax
import jax.numpy as jnp
from jax import lax
from jax.experimental import pallas as pl
from jax.experimental.pallas import tpu as pltpu
from jax.experimental.pallas import tpu_sc as plsc

D_MODEL = 64
SCALE = 8.0
NC, NS, LANES = 2, 16, 16
NW = NC * NS
CHUNK = 128
NBUF = 4
KGRP = D_MODEL // LANES


def _emb_body(x_hbm, table_hbm, out_hbm, idx_v, pidx, ibuf, sel, *sems):
    gsems = sems[:NBUF]
    ssems = sems[NBUF:]
    rpw = x_hbm.shape[0] // NW
    sid = lax.axis_index("s")
    wid = sid * NC + lax.axis_index("c")
    row0 = wid * rpw

    pltpu.sync_copy(x_hbm.at[pl.ds(row0, rpw)], idx_v)

    def fire_gather(cj, b):
        for g in range(CHUNK // LANES):
            iv = idx_v[cj, pl.ds(g * LANES, LANES)]
            pidx[b, pl.ds(g * LANES, LANES)] = iv >> 1
        pltpu.async_copy(table_hbm.at[pidx.at[b]], ibuf.at[b], gsems[b])

    def native_base(u):
        return 2048 * (u // 32) + 8 * (u % 32)

    for b in range(NBUF):
        fire_gather(b, b)

    @pl.loop(0, rpw, step=NBUF)
    def _(j):
        for b in range(NBUF):
            cj = j + b
            pltpu.make_async_copy(
                table_hbm.at[pidx.at[b]], ibuf.at[b], gsems[b]
            ).wait()

            @pl.when(cj >= NBUF)
            def _():
                up = row0 + cj - NBUF
                pltpu.make_async_copy(
                    sel.at[b], out_hbm.at[pl.ds(up * 64, 64)], ssems[b]
                ).wait()

            @pl.loop(0, CHUNK, step=LANES)
            def _(r):
                par_v = (idx_v[cj, pl.ds(r, LANES)] & 1) * D_MODEL
                for rr in range(LANES):
                    row = r + rr
                    half = par_v[rr]
                    for k in range(KGRP):
                        dst = pl.ds((row % 2) * D_MODEL + k * LANES, LANES)
                        sel[b, row // 2, dst] = (
                            ibuf[b, row, pl.ds(half + k * LANES, LANES)] * SCALE
                        )

            u = row0 + cj
            pltpu.async_copy(
                sel.at[b], out_hbm.at[pl.ds(u * 64, 64)], ssems[b]
            )

            @pl.when(cj + NBUF < rpw)
            def _():
                fire_gather(cj + NBUF, b)

    for b in range(NBUF):
        up = row0 + rpw - NBUF + b
        pltpu.make_async_copy(
            sel.at[b], out_hbm.at[pl.ds(up * 64, 64)], ssems[b]
        ).wait()


def kernel(x, table):
    b0, b1 = x.shape
    total = b0 * b1
    ntile = b0 // CHUNK
    xt = x.T.reshape(b1 * ntile, CHUNK)
    t2 = table.reshape(table.shape[0] // 2, 2 * D_MODEL)
    run = pl.kernel(
        _emb_body,
        out_type=jax.ShapeDtypeStruct((total // 2, 2 * D_MODEL), jnp.float32),
        mesh=plsc.VectorSubcoreMesh(core_axis_name="c", subcore_axis_name="s"),
        scratch_types=[
            pltpu.VMEM((total // CHUNK // NW, CHUNK), jnp.int32),
            pltpu.VMEM((NBUF, CHUNK), jnp.int32),
            pltpu.VMEM((NBUF, CHUNK, 2 * D_MODEL), jnp.float32),
            pltpu.VMEM((NBUF, CHUNK // 2, 2 * D_MODEL), jnp.float32),
        ]
        + [pltpu.SemaphoreType.DMA] * (2 * NBUF),
        compiler_params=pltpu.CompilerParams(use_tc_tiling_on_sc=False),
    )
    out2 = run(xt, t2)

    o5 = out2.reshape(b1, ntile, D_MODEL, 2, D_MODEL)
    out = o5.transpose(1, 2, 3, 0, 4)
    return out.reshape(b0, b1, D_MODEL)
